# Initial kernel scaffold; baseline (speedup 1.0000x reference)
import jax
import jax.numpy as jnp
from jax.experimental import pallas as pl


def kernel(x, edge_index, W_self1, W_neigh1, b1, W_self2, W_neigh2, b2):
    raise NotImplementedError("write your pallas kernel here")


# speedup vs baseline: 3.0793x; 3.0793x over previous
import jax
import jax.numpy as jnp
from jax import lax
from jax.experimental import pallas as pl
from jax.experimental.pallas import tpu as pltpu
from jax.experimental.pallas import tpu_sc as plsc

N = 10000
E = 320000
D = 128
NC = 2
NS = 16
NW = NC * NS
CHUNK = 128
KPT = 80
IB = 16
NBLK = KPT // IB
E_PAD = NW * KPT * CHUNK
N_ACC = 10112
RPT = N_ACC // NS
LANES = 16
ROWB = 1000


def _make_segsum(gather):
  mesh = plsc.VectorSubcoreMesh(core_axis_name="c", subcore_axis_name="s")
  out_type = [jax.ShapeDtypeStruct((NC, N_ACC, D), jnp.float32)]
  scratch = [
      pltpu.VMEM((IB, CHUNK), jnp.int32),
      pltpu.VMEM((IB, CHUNK), jnp.int32),
      pltpu.VMEM((CHUNK, D), jnp.float32),
      pltpu.VMEM_SHARED((N_ACC, D), jnp.float32),
      pltpu.SemaphoreType.DMA,
  ]

  def body(y_hbm, src_hbm, dst_hbm, agg_out, src_v, dst_v, rows_v, acc, sem):
    c = lax.axis_index("c")
    s = lax.axis_index("s")
    wid = c * NS + s

    z = jnp.zeros((LANES,), jnp.float32)

    def zrow(i, _):
      for k in range(D // LANES):
        rows_v[i, pl.ds(k * LANES, LANES)] = z
      return 0

    lax.fori_loop(0, CHUNK, zrow, 0)

    r0 = s * RPT
    for m in range(RPT // CHUNK):
      pltpu.sync_copy(rows_v, acc.at[pl.ds(r0 + m * CHUNK, CHUNK)])
    rem = RPT % CHUNK
    if rem:
      base = r0 + (RPT // CHUNK) * CHUNK
      pltpu.sync_copy(rows_v.at[pl.ds(0, rem)], acc.at[pl.ds(base, rem)])

    if not gather:
      one = jnp.ones((LANES,), jnp.float32)

      def orow(i, _):
        for k in range(D // LANES):
          rows_v[i, pl.ds(k * LANES, LANES)] = one
        return 0

      lax.fori_loop(0, CHUNK, orow, 0)

    plsc.subcore_barrier()

    def block(blk, _):
      if gather:
        pltpu.sync_copy(src_hbm.at[wid, pl.ds(blk * IB, IB)], src_v)
      pltpu.sync_copy(dst_hbm.at[wid, pl.ds(blk * IB, IB)], dst_v)

      def step(j, _):
        if gather:
          pltpu.async_copy(y_hbm.at[src_v.at[j]], rows_v, sem).wait()
        pltpu.sync_copy(rows_v, acc.at[dst_v.at[j]], add=True)
        return 0

      lax.fori_loop(0, IB, step, 0)
      return 0

    lax.fori_loop(0, NBLK, block, 0)
    plsc.subcore_barrier()

    pltpu.sync_copy(acc.at[pl.ds(r0, RPT)], agg_out.at[c, pl.ds(r0, RPT)])

  return pl.kernel(body, out_type=out_type, mesh=mesh, scratch_types=scratch,
                   name="segsum" if gather else "degcnt")


_segsum = _make_segsum(True)
_degcnt = _make_segsum(False)


def _mm2_body(x_ref, ws_ref, wn_ref, hs_ref, y_ref):
  xb = x_ref[...]
  hs_ref[...] = jnp.dot(xb, ws_ref[...], preferred_element_type=jnp.float32)
  y_ref[...] = jnp.dot(xb, wn_ref[...], preferred_element_type=jnp.float32)


_mm2 = pl.pallas_call(
    _mm2_body,
    grid=(N // ROWB,),
    in_specs=[
        pl.BlockSpec((ROWB, D), lambda i: (i, 0)),
        pl.BlockSpec((D, D), lambda i: (0, 0)),
        pl.BlockSpec((D, D), lambda i: (0, 0)),
    ],
    out_specs=[pl.BlockSpec((ROWB, D), lambda i: (i, 0))] * 2,
    out_shape=[jax.ShapeDtypeStruct((N, D), jnp.float32)] * 2,
)


def _combine1_body(hs_ref, aggp_ref, cntp_ref, b_ref, ws_ref, wn_ref,
                   hs2_ref, y2_ref):
  agg = aggp_ref[0] + aggp_ref[1]
  deg = cntp_ref[0, :, 0:1] + cntp_ref[1, :, 0:1]
  recip = 1.0 / jnp.maximum(deg, 1.0)
  h = hs_ref[...] + agg * recip + b_ref[...]
  h = jnp.maximum(h, 0.0)
  hs2_ref[...] = jnp.dot(h, ws_ref[...], preferred_element_type=jnp.float32)
  y2_ref[...] = jnp.dot(h, wn_ref[...], preferred_element_type=jnp.float32)


_combine1 = pl.pallas_call(
    _combine1_body,
    grid=(N // ROWB,),
    in_specs=[
        pl.BlockSpec((ROWB, D), lambda i: (i, 0)),
        pl.BlockSpec((NC, ROWB, D), lambda i: (0, i, 0)),
        pl.BlockSpec((NC, ROWB, D), lambda i: (0, i, 0)),
        pl.BlockSpec((1, D), lambda i: (0, 0)),
        pl.BlockSpec((D, D), lambda i: (0, 0)),
        pl.BlockSpec((D, D), lambda i: (0, 0)),
    ],
    out_specs=[pl.BlockSpec((ROWB, D), lambda i: (i, 0))] * 2,
    out_shape=[jax.ShapeDtypeStruct((N, D), jnp.float32)] * 2,
)


def _combine2_body(hs_ref, aggp_ref, cntp_ref, b_ref, out_ref):
  agg = aggp_ref[0] + aggp_ref[1]
  deg = cntp_ref[0, :, 0:1] + cntp_ref[1, :, 0:1]
  recip = 1.0 / jnp.maximum(deg, 1.0)
  out_ref[...] = hs_ref[...] + agg * recip + b_ref[...]


_combine2 = pl.pallas_call(
    _combine2_body,
    grid=(N // ROWB,),
    in_specs=[
        pl.BlockSpec((ROWB, D), lambda i: (i, 0)),
        pl.BlockSpec((NC, ROWB, D), lambda i: (0, i, 0)),
        pl.BlockSpec((NC, ROWB, D), lambda i: (0, i, 0)),
        pl.BlockSpec((1, D), lambda i: (0, 0)),
    ],
    out_specs=pl.BlockSpec((ROWB, D), lambda i: (i, 0)),
    out_shape=jax.ShapeDtypeStruct((N, D), jnp.float32),
)


def kernel(x, edge_index, W_self1, W_neigh1, b1, W_self2, W_neigh2, b2):
  src = edge_index[0]
  dst = edge_index[1]
  pad = E_PAD - E
  srcp = jnp.concatenate([src, jnp.zeros((pad,), jnp.int32)])
  dstp = jnp.concatenate([dst, jnp.full((pad,), N, jnp.int32)])
  srcp = srcp.reshape(NW, KPT, CHUNK)
  dstp = dstp.reshape(NW, KPT, CHUNK)

  hs1, y1 = _mm2(x, W_self1, W_neigh1)
  (cntp,) = _degcnt(x, srcp, dstp)
  (aggp1,) = _segsum(y1, srcp, dstp)
  hs2, y2 = _combine1(hs1, aggp1, cntp, b1.reshape(1, D), W_self2, W_neigh2)
  (aggp2,) = _segsum(y2, srcp, dstp)
  out = _combine2(hs2, aggp2, cntp, b2.reshape(1, D))
  return out


# speedup vs baseline: 3.2656x; 1.0605x over previous
import jax
import jax.numpy as jnp
from jax import lax
from jax.experimental import pallas as pl
from jax.experimental.pallas import tpu as pltpu
from jax.experimental.pallas import tpu_sc as plsc

N = 10000
E = 320000
D = 128
NC = 2
NS = 16
NW = NC * NS
CHUNK = 128
KPT = 80
IB = 16
NBLK = KPT // IB
E_PAD = NW * KPT * CHUNK
N_ACC = 10112
RPT = N_ACC // NS
LANES = 16
ROWB = 1000


def _make_segsum(gather):
  mesh = plsc.VectorSubcoreMesh(core_axis_name="c", subcore_axis_name="s")
  out_type = [jax.ShapeDtypeStruct((NC, N_ACC, D), jnp.float32)]
  nbuf = 2 if gather else 1
  scratch = [
      pltpu.VMEM((IB, CHUNK), jnp.int32),
      pltpu.VMEM((IB, CHUNK), jnp.int32),
      pltpu.VMEM((nbuf, CHUNK, D), jnp.float32),
      pltpu.VMEM_SHARED((N_ACC, D), jnp.float32),
      pltpu.SemaphoreType.DMA,
  ]

  def body(y_hbm, src_hbm, dst_hbm, agg_out, src_v, dst_v, rows_v, acc, sem):
    c = lax.axis_index("c")
    s = lax.axis_index("s")
    wid = c * NS + s

    z = jnp.zeros((LANES,), jnp.float32)

    def zrow(i, _):
      for k in range(D // LANES):
        rows_v[0, i, pl.ds(k * LANES, LANES)] = z
      return 0

    lax.fori_loop(0, CHUNK, zrow, 0)

    r0 = s * RPT
    for m in range(RPT // CHUNK):
      pltpu.sync_copy(rows_v.at[0], acc.at[pl.ds(r0 + m * CHUNK, CHUNK)])
    rem = RPT % CHUNK
    if rem:
      base = r0 + (RPT // CHUNK) * CHUNK
      pltpu.sync_copy(rows_v.at[0, pl.ds(0, rem)], acc.at[pl.ds(base, rem)])

    if not gather:
      one = jnp.ones((LANES,), jnp.float32)

      def orow(i, _):
        for k in range(D // LANES):
          rows_v[0, i, pl.ds(k * LANES, LANES)] = one
        return 0

      lax.fori_loop(0, CHUNK, orow, 0)

    plsc.subcore_barrier()

    def block(blk, _):
      if gather:
        pltpu.sync_copy(src_hbm.at[wid, pl.ds(blk * IB, IB)], src_v)
      pltpu.sync_copy(dst_hbm.at[wid, pl.ds(blk * IB, IB)], dst_v)

      if gather:
        pltpu.async_copy(y_hbm.at[src_v.at[0]], rows_v.at[0], sem)

        def step(j, _):
          p = lax.rem(j, 2)
          pltpu.make_async_copy(y_hbm.at[src_v.at[j]],
                                rows_v.at[p], sem).wait()

          @pl.when(j + 1 < IB)
          def _():
            pltpu.async_copy(y_hbm.at[src_v.at[j + 1]],
                             rows_v.at[lax.rem(j + 1, 2)], sem)

          pltpu.sync_copy(rows_v.at[p], acc.at[dst_v.at[j]], add=True)
          return 0
      else:

        def step(j, _):
          pltpu.sync_copy(rows_v.at[0], acc.at[dst_v.at[j]], add=True)
          return 0

      lax.fori_loop(0, IB, step, 0)
      return 0

    lax.fori_loop(0, NBLK, block, 0)
    plsc.subcore_barrier()

    pltpu.sync_copy(acc.at[pl.ds(r0, RPT)], agg_out.at[c, pl.ds(r0, RPT)])

  return pl.kernel(body, out_type=out_type, mesh=mesh, scratch_types=scratch,
                   name="segsum" if gather else "degcnt")


_segsum = _make_segsum(True)
_degcnt = _make_segsum(False)


def _mm2_body(x_ref, ws_ref, wn_ref, hs_ref, y_ref):
  xb = x_ref[...]
  hs_ref[...] = jnp.dot(xb, ws_ref[...], preferred_element_type=jnp.float32)
  y_ref[...] = jnp.dot(xb, wn_ref[...], preferred_element_type=jnp.float32)


_mm2 = pl.pallas_call(
    _mm2_body,
    grid=(N // ROWB,),
    in_specs=[
        pl.BlockSpec((ROWB, D), lambda i: (i, 0)),
        pl.BlockSpec((D, D), lambda i: (0, 0)),
        pl.BlockSpec((D, D), lambda i: (0, 0)),
    ],
    out_specs=[pl.BlockSpec((ROWB, D), lambda i: (i, 0))] * 2,
    out_shape=[jax.ShapeDtypeStruct((N, D), jnp.float32)] * 2,
)


def _combine1_body(hs_ref, aggp_ref, cntp_ref, b_ref, ws_ref, wn_ref,
                   hs2_ref, y2_ref):
  agg = aggp_ref[0] + aggp_ref[1]
  deg = cntp_ref[0, :, 0:1] + cntp_ref[1, :, 0:1]
  recip = 1.0 / jnp.maximum(deg, 1.0)
  h = hs_ref[...] + agg * recip + b_ref[...]
  h = jnp.maximum(h, 0.0)
  hs2_ref[...] = jnp.dot(h, ws_ref[...], preferred_element_type=jnp.float32)
  y2_ref[...] = jnp.dot(h, wn_ref[...], preferred_element_type=jnp.float32)


_combine1 = pl.pallas_call(
    _combine1_body,
    grid=(N // ROWB,),
    in_specs=[
        pl.BlockSpec((ROWB, D), lambda i: (i, 0)),
        pl.BlockSpec((NC, ROWB, D), lambda i: (0, i, 0)),
        pl.BlockSpec((NC, ROWB, D), lambda i: (0, i, 0)),
        pl.BlockSpec((1, D), lambda i: (0, 0)),
        pl.BlockSpec((D, D), lambda i: (0, 0)),
        pl.BlockSpec((D, D), lambda i: (0, 0)),
    ],
    out_specs=[pl.BlockSpec((ROWB, D), lambda i: (i, 0))] * 2,
    out_shape=[jax.ShapeDtypeStruct((N, D), jnp.float32)] * 2,
)


def _combine2_body(hs_ref, aggp_ref, cntp_ref, b_ref, out_ref):
  agg = aggp_ref[0] + aggp_ref[1]
  deg = cntp_ref[0, :, 0:1] + cntp_ref[1, :, 0:1]
  recip = 1.0 / jnp.maximum(deg, 1.0)
  out_ref[...] = hs_ref[...] + agg * recip + b_ref[...]


_combine2 = pl.pallas_call(
    _combine2_body,
    grid=(N // ROWB,),
    in_specs=[
        pl.BlockSpec((ROWB, D), lambda i: (i, 0)),
        pl.BlockSpec((NC, ROWB, D), lambda i: (0, i, 0)),
        pl.BlockSpec((NC, ROWB, D), lambda i: (0, i, 0)),
        pl.BlockSpec((1, D), lambda i: (0, 0)),
    ],
    out_specs=pl.BlockSpec((ROWB, D), lambda i: (i, 0)),
    out_shape=jax.ShapeDtypeStruct((N, D), jnp.float32),
)


def kernel(x, edge_index, W_self1, W_neigh1, b1, W_self2, W_neigh2, b2):
  src = edge_index[0]
  dst = edge_index[1]
  pad = E_PAD - E
  srcp = jnp.concatenate([src, jnp.zeros((pad,), jnp.int32)])
  dstp = jnp.concatenate([dst, jnp.full((pad,), N, jnp.int32)])
  srcp = srcp.reshape(NW, KPT, CHUNK)
  dstp = dstp.reshape(NW, KPT, CHUNK)

  hs1, y1 = _mm2(x, W_self1, W_neigh1)
  (cntp,) = _degcnt(x, srcp, dstp)
  (aggp1,) = _segsum(y1, srcp, dstp)
  hs2, y2 = _combine1(hs1, aggp1, cntp, b1.reshape(1, D), W_self2, W_neigh2)
  (aggp2,) = _segsum(y2, srcp, dstp)
  out = _combine2(hs2, aggp2, cntp, b2.reshape(1, D))
  return out
